# baseline (speedup 1.0000x reference)
import jax
import jax.numpy as jnp
from jax import lax
from jax.experimental import pallas as pl
from jax.experimental.pallas import tpu as pltpu
from jax.experimental.pallas import tpu_sc as plsc

EMBED = 64
SRC_LEN = 200
Q_USED = 36
BATCH = 1024
SEQ = Q_USED + SRC_LEN
TOTAL_ROWS = BATCH * SEQ
NC, NS = 2, 16
NW = NC * NS
BPW = BATCH // NW
ROWS_PW = TOTAL_ROWS // NW
PAIRS_PW = ROWS_PW // 2
PCHUNK = 236
PCHUNK_PAD = 240
NBCH = PAIRS_PW // PCHUNK
NVREG = (SEQ + 15) // 16


def _gather_body(story_hbm, question_hbm, table_hbm, out_hbm,
                 qv, sv, idx_e, idx_o, bufs_e0, bufs_e1, bufs_o0, bufs_o1,
                 sem_g0, sem_g1, sem_w0, sem_w1):
    wid = lax.axis_index("s") * NC + lax.axis_index("c")
    b0 = wid * BPW
    h0 = wid * PAIRS_PW

    pltpu.sync_copy(question_hbm.at[pl.ds(0, Q_USED), pl.ds(b0, BPW)], qv)
    pltpu.sync_copy(story_hbm.at[:, pl.ds(b0, BPW)], sv)

    lane = lax.iota(jnp.int32, 16)

    zero16 = jnp.zeros((16,), jnp.int32)
    for j in range(PCHUNK_PAD - PCHUNK):
        cpad = jnp.full((16,), PCHUNK + j, jnp.int32)
        plsc.store_scatter(idx_e, [lane, cpad], zero16)
        plsc.store_scatter(idx_o, [lane, cpad], zero16)

    def store_idx(x, t_vec, b, valid):
        p = b * SEQ + t_vec
        h = p // 2
        r = h // PCHUNK
        c = h - r * PCHUNK
        m_e = (p & 1) == 0
        m_o = (p & 1) == 1
        if valid is not None:
            m_e = m_e & valid
            m_o = m_o & valid
        plsc.store_scatter(idx_e, [r, c], x, mask=m_e)
        plsc.store_scatter(idx_o, [r, c], x, mask=m_o)

    def tr_body(b, carry):
        b_vec = jnp.full((16,), b, jnp.int32)
        for k in range(NVREG):
            t_vec = lane + (16 * k)
            if 16 * (k + 1) <= Q_USED:
                x = plsc.load_gather(qv, [t_vec, b_vec])
                store_idx(x, t_vec, b, None)
            elif 16 * k >= Q_USED and 16 * (k + 1) <= SEQ:
                x = plsc.load_gather(sv, [t_vec - Q_USED, b_vec])
                store_idx(x, t_vec, b, None)
            elif 16 * k < Q_USED:
                mq = t_vec < Q_USED
                xq = plsc.load_gather(qv, [jnp.where(mq, t_vec, 0), b_vec],
                                      mask=mq)
                xs = plsc.load_gather(
                    sv, [jnp.where(mq, 0, t_vec - Q_USED), b_vec], mask=~mq)
                store_idx(jnp.where(mq, xq, xs), t_vec, b, None)
            else:
                mt = t_vec < SEQ
                x = plsc.load_gather(
                    sv, [jnp.where(mt, t_vec - Q_USED, 0), b_vec], mask=mt)
                store_idx(x, t_vec, b, mt)
        return carry

    lax.fori_loop(0, BPW, tr_body, 0)

    bufs_e = (bufs_e0, bufs_e1)
    bufs_o = (bufs_o0, bufs_o1)
    sem_g = (sem_g0, sem_g1)
    sem_w = (sem_w0, sem_w1)
    gde = [None] * NBCH
    gdo = [None] * NBCH
    wde = [None] * NBCH
    wdo = [None] * NBCH

    def start_gather(g):
        s = g % 2
        gde[g] = pltpu.async_copy(
            table_hbm.at[idx_e.at[g]], bufs_e[s], sem_g[s])
        gdo[g] = pltpu.async_copy(
            table_hbm.at[idx_o.at[g]], bufs_o[s], sem_g[s])

    start_gather(0)
    for g in range(NBCH):
        s = g % 2
        gde[g].wait()
        gdo[g].wait()
        if g >= 1:
            wde[g - 1].wait()
            wdo[g - 1].wait()
        if g < NBCH - 1:
            start_gather(g + 1)
        rows = pl.ds(h0 + g * PCHUNK, PCHUNK)
        wde[g] = pltpu.async_copy(
            bufs_e[s].at[pl.ds(0, PCHUNK)],
            out_hbm.at[rows, pl.ds(0, EMBED)], sem_w[s])
        wdo[g] = pltpu.async_copy(
            bufs_o[s].at[pl.ds(0, PCHUNK)],
            out_hbm.at[rows, pl.ds(EMBED, EMBED)], sem_w[s])
    wde[NBCH - 1].wait()
    wdo[NBCH - 1].wait()


def kernel(story, question, embed_table):
    mesh = plsc.VectorSubcoreMesh(
        core_axis_name="c", subcore_axis_name="s",
        num_cores=NC, num_subcores=NS,
    )
    out = pl.kernel(
        _gather_body,
        out_type=jax.ShapeDtypeStruct((TOTAL_ROWS // 2, 2 * EMBED),
                                      jnp.float32),
        mesh=mesh,
        scratch_types=[
            pltpu.VMEM((Q_USED, BPW), jnp.int32),
            pltpu.VMEM((SRC_LEN, BPW), jnp.int32),
            pltpu.VMEM((NBCH, PCHUNK_PAD), jnp.int32),
            pltpu.VMEM((NBCH, PCHUNK_PAD), jnp.int32),
            pltpu.VMEM((PCHUNK_PAD, EMBED), jnp.float32),
            pltpu.VMEM((PCHUNK_PAD, EMBED), jnp.float32),
            pltpu.VMEM((PCHUNK_PAD, EMBED), jnp.float32),
            pltpu.VMEM((PCHUNK_PAD, EMBED), jnp.float32),
            pltpu.SemaphoreType.DMA,
            pltpu.SemaphoreType.DMA,
            pltpu.SemaphoreType.DMA,
            pltpu.SemaphoreType.DMA,
        ],
        compiler_params=pltpu.CompilerParams(
            use_tc_tiling_on_sc=False, needs_layout_passes=False),
    )(story.astype(jnp.int32), question.astype(jnp.int32), embed_table)
    return out.reshape(BATCH, SEQ, EMBED)
